# baseline (speedup 1.0000x reference)
import functools

import jax
import jax.numpy as jnp
from jax import lax
from jax.experimental import pallas as pl
from jax.experimental.pallas import tpu as pltpu
from jax.experimental.pallas import tpu_sc as plsc

N = 50000
K = 9
CH = 128
M = 65536
NP = 51200
NW = 32
NS = 16


def _worker_id():
    return lax.axis_index("c") * NS + lax.axis_index("s")


def _sc_gather_x3(xT, idxs, *, name):
    nch_tab, tab_n, rpw = xT.shape[0], xT.shape[1], idxs.shape[0] // NW
    assert tab_n % 8 == 0 and rpw % 16 == 0
    mesh = plsc.VectorSubcoreMesh(core_axis_name="c", subcore_axis_name="s")

    @functools.partial(
        pl.kernel,
        out_type=jax.ShapeDtypeStruct((NW * rpw * nch_tab,), jnp.float32),
        mesh=mesh,
        scratch_types=[
            pltpu.VMEM((tab_n,), jnp.float32),
            pltpu.VMEM((rpw,), jnp.int32),
            pltpu.VMEM((rpw * nch_tab,), jnp.float32),
        ],
        name=name,
        compiler_params=pltpu.CompilerParams(needs_layout_passes=False),
    )
    def k(x_h, idx_h, out_h, tab_v, idx_v, out_v):
        w = _worker_id()
        rln = rpw // K
        npt = NP
        iota = lax.iota(jnp.int32, 16)
        pltpu.sync_copy(idx_h.at[pl.ds(w * rpw, rpw)], idx_v)
        for c in range(nch_tab):
            pltpu.sync_copy(x_h.at[pl.ds(c * tab_n, tab_n)], tab_v)

            def step(v, carry):
                iv = idx_v[pl.ds(v * 16, 16)]
                vals = plsc.load_gather(tab_v, [iv])
                p = iota + v * 16
                kk = lax.rem(p, K)
                nl = lax.div(p, K)
                plsc.store_scatter(
                    out_v, [(kk * nch_tab + c) * rln + nl], vals)
                return carry

            lax.fori_loop(0, rpw // 16, step, 0)
        for j in range(K * nch_tab):
            pltpu.sync_copy(out_v.at[pl.ds(j * rln, rln)],
                            out_h.at[pl.ds(j * npt + w * rln, rln)])

    return k(xT.reshape(-1), idxs)


def _sc_gather_group(table, idxs, *, grp, iw, spb, nch, relu, scale,
                     mask_n, name, packed=False):
    T, D = table.shape
    do = D * 2 if packed else D
    rtot = idxs.shape[0]
    ch = spb * iw
    assert ch % grp == 0
    cn = ch // grp
    rpw = rtot // NW
    assert rpw == nch * ch and nch % 2 == 0
    orows = rtot // grp
    opw = orows // NW
    hb = D // 16
    mesh = plsc.VectorSubcoreMesh(core_axis_name="c", subcore_axis_name="s")

    @functools.partial(
        pl.kernel,
        out_type=jax.ShapeDtypeStruct((orows, do), jnp.float32),
        mesh=mesh,
        scratch_types=[
            pltpu.VMEM((rpw,), jnp.int32),
            pltpu.VMEM((ch, D), table.dtype),
            pltpu.VMEM((ch, D), table.dtype),
            pltpu.VMEM((cn, do), jnp.float32),
            pltpu.VMEM((cn, do), jnp.float32),
            pltpu.SemaphoreType.DMA,
            pltpu.SemaphoreType.DMA,
            pltpu.SemaphoreType.DMA,
            pltpu.SemaphoreType.DMA,
        ],
        name=name,
        compiler_params=pltpu.CompilerParams(
            needs_layout_passes=False,
            use_tc_tiling_on_sc=not packed),
    )
    def k(table_h, idx_h, out_h, idx_v, buf0, buf1, ov0, ov1,
          sg0, sg1, so0, so1):
        w = _worker_id()
        pltpu.sync_copy(idx_h.at[pl.ds(w * rpw, rpw)], idx_v)
        bufs = (buf0, buf1)
        ovs = (ov0, ov1)
        sgs = (sg0, sg1)
        sos = (so0, so1)

        def issue_gather(g, b):
            for j in range(spb):
                pltpu.async_copy(
                    table_h.at[idx_v.at[pl.ds((g * spb + j) * iw, iw)]],
                    bufs[b].at[pl.ds(j * iw, iw)],
                    sgs[b],
                )

        def wait_gather(b):
            for j in range(spb):
                pltpu.make_async_copy(
                    table_h.at[idx_v.at[pl.ds(0, iw)]],
                    bufs[b].at[pl.ds(j * iw, iw)],
                    sgs[b],
                ).wait()

        def issue_store(g, b):
            pltpu.async_copy(
                ovs[b], out_h.at[pl.ds(w * opw + g * cn, cn)], sos[b]
            )

        def wait_store(b):
            pltpu.make_async_copy(
                ovs[b], out_h.at[pl.ds(w * opw, cn)], sos[b]
            ).wait()

        issue_gather(0, 0)

        def pair(p, carry):
            for b in (0, 1):
                g = p * 2 + b
                if b == 0:
                    issue_gather(g + 1, 1)
                else:

                    @pl.when(p < nch // 2 - 1)
                    def _():
                        issue_gather(g + 1, 0)

                @pl.when(p >= 1)
                def _():
                    wait_store(b)

                wait_gather(b)

                def node(i, c):
                    row0 = i * grp
                    if mask_n is not None:
                        gn = w * opw + g * cn + i
                        f = jnp.where(gn < mask_n, 1.0, 0.0).astype(
                            jnp.float32)

                    def post(acc):
                        if scale is not None:
                            acc = acc * scale
                        if relu:
                            acc = jnp.maximum(acc, 0.0)
                        if mask_n is not None:
                            acc = acc * f
                        return acc

                    if packed:
                        for h in range(D // 16):
                            sl = pl.ds(h * 16, 16)
                            lo = hi = None
                            for kk in range(grp):
                                wv = bufs[b][row0 + kk, sl]
                                lof = plsc.bitcast(
                                    lax.shift_left(wv, 16), jnp.float32)
                                hif = plsc.bitcast(
                                    wv & jnp.int32(-65536), jnp.float32)
                                lo = lof if lo is None else lo + lof
                                hi = hif if hi is None else hi + hif
                            ovs[b][i, pl.ds(h * 16, 16)] = post(lo)
                            ovs[b][i, pl.ds(D + h * 16, 16)] = post(hi)
                    else:
                        for h in range(hb):
                            sl = pl.ds(h * 16, 16)
                            acc = bufs[b][row0, sl]
                            for kk in range(1, grp):
                                acc = acc + bufs[b][row0 + kk, sl]
                            ovs[b][i, sl] = post(acc)
                    return c

                lax.fori_loop(0, cn, node, 0)
                issue_store(g, b)
            return carry

        lax.fori_loop(0, nch // 2, pair, 0)
        wait_store(0)
        wait_store(1)

    return k(table, idxs)


def _tc_convs(g1t, w1p, b1, w2, b2):
    npd = g1t.shape[1]
    bn = 512
    grid = npd // bn
    cin = g1t.shape[0]

    def body(g1_ref, w1_ref, b1_ref, w2_ref, b2_ref, out_ref):
        h = lax.dot_general(g1_ref[...], w1_ref[...],
                            (((0,), (0,)), ((), ())),
                            preferred_element_type=jnp.float32)
        h = jnp.maximum(h + b1_ref[...], 0.0)
        for k in range(K):
            r = jnp.dot(h, w2_ref[k], preferred_element_type=jnp.float32)
            if k == 0:
                r = r + b2_ref[...]
            lo = lax.bitcast_convert_type(
                r[:, :CH // 2].astype(jnp.bfloat16), jnp.uint16)
            hi = lax.bitcast_convert_type(
                r[:, CH // 2:].astype(jnp.bfloat16), jnp.uint16)
            word = (lo.astype(jnp.uint32)
                    | (hi.astype(jnp.uint32) << 16))
            out_ref[k] = lax.bitcast_convert_type(word, jnp.int32)

    return pl.pallas_call(
        body,
        grid=(grid,),
        in_specs=[
            pl.BlockSpec((cin, bn), lambda i: (0, i)),
            pl.BlockSpec((cin, CH), lambda i: (0, 0)),
            pl.BlockSpec((1, CH), lambda i: (0, 0)),
            pl.BlockSpec((K, CH, CH), lambda i: (0, 0, 0)),
            pl.BlockSpec((1, CH), lambda i: (0, 0)),
        ],
        out_specs=pl.BlockSpec((K, bn, CH // 2), lambda i: (0, i, 0)),
        out_shape=jax.ShapeDtypeStruct((K, npd, CH // 2), jnp.int32),
    )(g1t, w1p, b1.reshape(1, CH), w2, b2.reshape(1, CH))


def kernel(x, adj, perm, W1, b1, W2, b2):
    xT = x[0].T
    spread = (jnp.arange((NP - N) * K, dtype=jnp.int32) % N).reshape(-1, K)
    adjp = jnp.concatenate([adj, spread], axis=0)
    idx1 = adjp.reshape(-1)
    g1 = _sc_gather_x3(xT, idx1, name="sc_gather_x")
    g1t = g1.reshape(K * 3, NP)
    t2 = _tc_convs(g1t, W1.reshape(K * 3, CH), b1, W2, b2)
    t2f = t2.reshape(K * NP, CH // 2)
    idx2 = (adjp + NP * jnp.arange(K, dtype=jnp.int32)[None, :]).reshape(-1)
    out2 = _sc_gather_group(t2f, idx2, grp=K, iw=96, spb=3, nch=50,
                            relu=True, scale=None, mask_n=N,
                            name="sc_conv2_gsum", packed=True)
    permc = jnp.where(perm < N, perm, N + (perm - N) % (NP - N))
    pooled = _sc_gather_group(out2, permc.astype(jnp.int32), grp=4, iw=64,
                              spb=4, nch=8, relu=False, scale=0.25,
                              mask_n=None, name="sc_pool")
    return pooled[None]


# speedup vs baseline: 1.1968x; 1.1968x over previous
import functools

import jax
import jax.numpy as jnp
from jax import lax
from jax.experimental import pallas as pl
from jax.experimental.pallas import tpu as pltpu
from jax.experimental.pallas import tpu_sc as plsc

N = 50000
K = 9
CH = 128
M = 65536
NP = 51200
NW = 32
NS = 16


def _worker_id():
    return lax.axis_index("c") * NS + lax.axis_index("s")


def _sc_gather_x3(xT, idxs, *, name):
    nch_tab, tab_n, rpw = xT.shape[0], xT.shape[1], idxs.shape[0] // NW
    assert tab_n % 8 == 0 and rpw % 16 == 0
    mesh = plsc.VectorSubcoreMesh(core_axis_name="c", subcore_axis_name="s")

    @functools.partial(
        pl.kernel,
        out_type=jax.ShapeDtypeStruct((NW * rpw * nch_tab,), jnp.float32),
        mesh=mesh,
        scratch_types=[
            pltpu.VMEM((tab_n,), jnp.float32),
            pltpu.VMEM((rpw,), jnp.int32),
            pltpu.VMEM((rpw * nch_tab,), jnp.float32),
        ],
        name=name,
        compiler_params=pltpu.CompilerParams(needs_layout_passes=False),
    )
    def k(x_h, idx_h, out_h, tab_v, idx_v, out_v):
        w = _worker_id()
        rln = rpw // K
        npt = NP
        iota = lax.iota(jnp.int32, 16)
        pltpu.sync_copy(idx_h.at[pl.ds(w * rpw, rpw)], idx_v)
        for c in range(nch_tab):
            pltpu.sync_copy(x_h.at[pl.ds(c * tab_n, tab_n)], tab_v)

            def step(v, carry):
                iv = idx_v[pl.ds(v * 16, 16)]
                vals = plsc.load_gather(tab_v, [iv])
                p = iota + v * 16
                kk = lax.rem(p, K)
                nl = lax.div(p, K)
                plsc.store_scatter(
                    out_v, [(kk * nch_tab + c) * rln + nl], vals)
                return carry

            lax.fori_loop(0, rpw // 16, step, 0)
        for j in range(K * nch_tab):
            pltpu.sync_copy(out_v.at[pl.ds(j * rln, rln)],
                            out_h.at[pl.ds(j * npt + w * rln, rln)])

    return k(xT.reshape(-1), idxs)


def _sc_gather_group(table, idxs, *, grp, iw, spb, nch, relu, scale,
                     mask_n, name, packed=False):
    T, D = table.shape
    do = D * 2 if packed else D
    rtot = idxs.shape[0]
    ch = spb * iw
    assert ch % grp == 0
    cn = ch // grp
    rpw = rtot // NW
    assert rpw == nch * ch and nch % 2 == 0
    orows = rtot // grp
    opw = orows // NW
    hb = D // 16
    mesh = plsc.VectorSubcoreMesh(core_axis_name="c", subcore_axis_name="s")

    @functools.partial(
        pl.kernel,
        out_type=jax.ShapeDtypeStruct((orows, do), jnp.float32),
        mesh=mesh,
        scratch_types=[
            pltpu.VMEM((rpw,), jnp.int32),
            pltpu.VMEM((ch, D), table.dtype),
            pltpu.VMEM((ch, D), table.dtype),
            pltpu.VMEM((cn, do), jnp.float32),
            pltpu.VMEM((cn, do), jnp.float32),
            pltpu.SemaphoreType.DMA,
            pltpu.SemaphoreType.DMA,
            pltpu.SemaphoreType.DMA,
            pltpu.SemaphoreType.DMA,
        ],
        name=name,
        compiler_params=pltpu.CompilerParams(needs_layout_passes=False),
    )
    def k(table_h, idx_h, out_h, idx_v, buf0, buf1, ov0, ov1,
          sg0, sg1, so0, so1):
        w = _worker_id()
        pltpu.sync_copy(idx_h.at[pl.ds(w * rpw, rpw)], idx_v)
        bufs = (buf0, buf1)
        ovs = (ov0, ov1)
        sgs = (sg0, sg1)
        sos = (so0, so1)

        def issue_gather(g, b):
            for j in range(spb):
                pltpu.async_copy(
                    table_h.at[idx_v.at[pl.ds((g * spb + j) * iw, iw)]],
                    bufs[b].at[pl.ds(j * iw, iw)],
                    sgs[b],
                )

        def wait_gather(b):
            for j in range(spb):
                pltpu.make_async_copy(
                    table_h.at[idx_v.at[pl.ds(0, iw)]],
                    bufs[b].at[pl.ds(j * iw, iw)],
                    sgs[b],
                ).wait()

        def issue_store(g, b):
            pltpu.async_copy(
                ovs[b], out_h.at[pl.ds(w * opw + g * cn, cn)], sos[b]
            )

        def wait_store(b):
            pltpu.make_async_copy(
                ovs[b], out_h.at[pl.ds(w * opw, cn)], sos[b]
            ).wait()

        issue_gather(0, 0)

        def pair(p, carry):
            for b in (0, 1):
                g = p * 2 + b
                if b == 0:
                    issue_gather(g + 1, 1)
                else:

                    @pl.when(p < nch // 2 - 1)
                    def _():
                        issue_gather(g + 1, 0)

                @pl.when(p >= 1)
                def _():
                    wait_store(b)

                wait_gather(b)

                def node(i, c):
                    row0 = i * grp
                    if mask_n is not None:
                        gn = w * opw + g * cn + i
                        f = jnp.where(gn < mask_n, 1.0, 0.0).astype(
                            jnp.float32)

                    def post(acc):
                        if scale is not None:
                            acc = acc * scale
                        if relu:
                            acc = jnp.maximum(acc, 0.0)
                        if mask_n is not None:
                            acc = acc * f
                        return acc

                    if packed:
                        for h in range(D // 16):
                            sl = pl.ds(h * 16, 16)
                            lo = hi = None
                            for kk in range(grp):
                                wv = bufs[b][row0 + kk, sl]
                                lof = plsc.bitcast(
                                    lax.shift_left(wv, 16), jnp.float32)
                                hif = plsc.bitcast(
                                    wv & jnp.int32(-65536), jnp.float32)
                                lo = lof if lo is None else lo + lof
                                hi = hif if hi is None else hi + hif
                            ovs[b][i, pl.ds(h * 16, 16)] = post(lo)
                            ovs[b][i, pl.ds(D + h * 16, 16)] = post(hi)
                    else:
                        for h in range(hb):
                            sl = pl.ds(h * 16, 16)
                            acc = bufs[b][row0, sl]
                            for kk in range(1, grp):
                                acc = acc + bufs[b][row0 + kk, sl]
                            ovs[b][i, sl] = post(acc)
                    return c

                lax.fori_loop(0, cn, node, 0)
                issue_store(g, b)
            return carry

        lax.fori_loop(0, nch // 2, pair, 0)
        wait_store(0)
        wait_store(1)

    return k(table, idxs)


def _tc_convs(g1t, w1p, b1, w2, b2):
    npd = g1t.shape[1]
    bn = 512
    grid = npd // bn
    cin = g1t.shape[0]

    def body(g1_ref, w1_ref, b1_ref, w2_ref, b2_ref, out_ref):
        h = lax.dot_general(g1_ref[...], w1_ref[...],
                            (((0,), (0,)), ((), ())),
                            preferred_element_type=jnp.float32)
        h = jnp.maximum(h + b1_ref[...], 0.0)
        for k in range(K):
            r = jnp.dot(h, w2_ref[k], preferred_element_type=jnp.float32)
            if k == 0:
                r = r + b2_ref[...]
            out_ref[k] = r

    return pl.pallas_call(
        body,
        grid=(grid,),
        in_specs=[
            pl.BlockSpec((cin, bn), lambda i: (0, i)),
            pl.BlockSpec((cin, CH), lambda i: (0, 0)),
            pl.BlockSpec((1, CH), lambda i: (0, 0)),
            pl.BlockSpec((K, CH, CH), lambda i: (0, 0, 0)),
            pl.BlockSpec((1, CH), lambda i: (0, 0)),
        ],
        out_specs=pl.BlockSpec((K, bn, CH), lambda i: (0, i, 0)),
        out_shape=jax.ShapeDtypeStruct((K, npd, CH), jnp.float32),
    )(g1t, w1p, b1.reshape(1, CH), w2, b2.reshape(1, CH))


def kernel(x, adj, perm, W1, b1, W2, b2):
    xT = x[0].T
    spread = (jnp.arange((NP - N) * K, dtype=jnp.int32) % N).reshape(-1, K)
    adjp = jnp.concatenate([adj, spread], axis=0)
    idx1 = adjp.reshape(-1)
    g1 = _sc_gather_x3(xT, idx1, name="sc_gather_x")
    g1t = g1.reshape(K * 3, NP)
    t2 = _tc_convs(g1t, W1.reshape(K * 3, CH), b1, W2, b2)
    t2f = t2.reshape(K * NP, CH)
    idx2 = (adjp + NP * jnp.arange(K, dtype=jnp.int32)[None, :]).reshape(-1)
    out2 = _sc_gather_group(t2f, idx2, grp=K, iw=96, spb=3, nch=50,
                            relu=True, scale=None, mask_n=N,
                            name="sc_conv2_gsum")
    permc = jnp.where(perm < N, perm, N + (perm - N) % (NP - N))
    pooled = _sc_gather_group(out2, permc.astype(jnp.int32), grp=4, iw=64,
                              spb=4, nch=8, relu=False, scale=0.25,
                              mask_n=None, name="sc_pool")
    return pooled[None]


# speedup vs baseline: 1.1983x; 1.0013x over previous
import functools

import jax
import jax.numpy as jnp
from jax import lax
from jax.experimental import pallas as pl
from jax.experimental.pallas import tpu as pltpu
from jax.experimental.pallas import tpu_sc as plsc

N = 50000
K = 9
CH = 128
M = 65536
NP = 51200
NW = 32
NS = 16


def _worker_id():
    return lax.axis_index("c") * NS + lax.axis_index("s")


def _sc_gather_x3(xT, idxs, *, name):
    nch_tab, tab_n, rpw = xT.shape[0], xT.shape[1], idxs.shape[0] // NW
    assert tab_n % 8 == 0 and rpw % 16 == 0
    mesh = plsc.VectorSubcoreMesh(core_axis_name="c", subcore_axis_name="s")

    @functools.partial(
        pl.kernel,
        out_type=jax.ShapeDtypeStruct((NW * rpw * nch_tab,), jnp.float32),
        mesh=mesh,
        scratch_types=[
            pltpu.VMEM((tab_n,), jnp.float32),
            pltpu.VMEM((rpw,), jnp.int32),
            pltpu.VMEM((rpw * nch_tab,), jnp.float32),
        ],
        name=name,
        compiler_params=pltpu.CompilerParams(needs_layout_passes=False),
    )
    def k(x_h, idx_h, out_h, tab_v, idx_v, out_v):
        w = _worker_id()
        rln = rpw // K
        npt = NP
        iota = lax.iota(jnp.int32, 16)
        pltpu.sync_copy(idx_h.at[pl.ds(w * rpw, rpw)], idx_v)
        for c in range(nch_tab):
            pltpu.sync_copy(x_h.at[pl.ds(c * tab_n, tab_n)], tab_v)

            def step(v, carry):
                iv = idx_v[pl.ds(v * 16, 16)]
                vals = plsc.load_gather(tab_v, [iv])
                p = iota + v * 16
                kk = lax.rem(p, K)
                nl = lax.div(p, K)
                plsc.store_scatter(
                    out_v, [(kk * nch_tab + c) * rln + nl], vals)
                return carry

            lax.fori_loop(0, rpw // 16, step, 0)
        for j in range(K * nch_tab):
            pltpu.sync_copy(out_v.at[pl.ds(j * rln, rln)],
                            out_h.at[pl.ds(j * npt + w * rln, rln)])

    return k(xT.reshape(-1), idxs)


def _sc_gather_group(table, idxs, *, grp, iw, spb, nch, relu, scale,
                     mask_n, name):
    T, D = table.shape
    rtot = idxs.shape[0]
    ch = spb * iw
    assert ch % grp == 0
    cn = ch // grp
    rpw = rtot // NW
    assert rpw == nch * ch and nch % 2 == 0
    orows = rtot // grp
    opw = orows // NW
    hb = D // 16
    mesh = plsc.VectorSubcoreMesh(core_axis_name="c", subcore_axis_name="s")

    @functools.partial(
        pl.kernel,
        out_type=jax.ShapeDtypeStruct((orows, D), jnp.float32),
        mesh=mesh,
        scratch_types=[
            pltpu.VMEM((rpw,), jnp.int32),
            pltpu.VMEM((ch, D), jnp.float32),
            pltpu.VMEM((ch, D), jnp.float32),
            pltpu.VMEM((cn, D), jnp.float32),
            pltpu.VMEM((cn, D), jnp.float32),
            pltpu.SemaphoreType.DMA,
            pltpu.SemaphoreType.DMA,
            pltpu.SemaphoreType.DMA,
            pltpu.SemaphoreType.DMA,
        ],
        name=name,
        compiler_params=pltpu.CompilerParams(needs_layout_passes=False),
    )
    def k(table_h, idx_h, out_h, idx_v, buf0, buf1, ov0, ov1,
          sg0, sg1, so0, so1):
        w = _worker_id()
        pltpu.sync_copy(idx_h.at[pl.ds(w * rpw, rpw)], idx_v)
        bufs = (buf0, buf1)
        ovs = (ov0, ov1)
        sgs = (sg0, sg1)
        sos = (so0, so1)

        def issue_gather(g, b):
            for j in range(spb):
                pltpu.async_copy(
                    table_h.at[idx_v.at[pl.ds((g * spb + j) * iw, iw)]],
                    bufs[b].at[pl.ds(j * iw, iw)],
                    sgs[b],
                )

        def wait_gather(b):
            for j in range(spb):
                pltpu.make_async_copy(
                    table_h.at[idx_v.at[pl.ds(0, iw)]],
                    bufs[b].at[pl.ds(j * iw, iw)],
                    sgs[b],
                ).wait()

        def issue_store(g, b):
            pltpu.async_copy(
                ovs[b], out_h.at[pl.ds(w * opw + g * cn, cn)], sos[b]
            )

        def wait_store(b):
            pltpu.make_async_copy(
                ovs[b], out_h.at[pl.ds(w * opw, cn)], sos[b]
            ).wait()

        issue_gather(0, 0)

        def pair(p, carry):
            for b in (0, 1):
                g = p * 2 + b
                if b == 0:
                    issue_gather(g + 1, 1)
                else:

                    @pl.when(p < nch // 2 - 1)
                    def _():
                        issue_gather(g + 1, 0)

                @pl.when(p >= 1)
                def _():
                    wait_store(b)

                wait_gather(b)

                def node(i, c):
                    row0 = i * grp
                    if mask_n is not None:
                        gn = w * opw + g * cn + i
                        f = jnp.where(gn < mask_n, 1.0, 0.0).astype(
                            jnp.float32)

                    def post(acc):
                        if scale is not None:
                            acc = acc * scale
                        if relu:
                            acc = jnp.maximum(acc, 0.0)
                        if mask_n is not None:
                            acc = acc * f
                        return acc

                    for h in range(hb):
                        sl = pl.ds(h * 16, 16)
                        acc = bufs[b][row0, sl]
                        for kk in range(1, grp):
                            acc = acc + bufs[b][row0 + kk, sl]
                        ovs[b][i, sl] = post(acc)
                    return c

                lax.fori_loop(0, cn, node, 0)
                issue_store(g, b)
            return carry

        lax.fori_loop(0, nch // 2, pair, 0)
        wait_store(0)
        wait_store(1)

    return k(table, idxs)


def _tc_convs(g1t, w1p, b1, w2, b2):
    npd = g1t.shape[1]
    bn = 512
    grid = npd // bn
    cin = g1t.shape[0]

    def body(g1_ref, w1_ref, b1_ref, w2_ref, b2_ref, out_ref):
        h = lax.dot_general(g1_ref[...], w1_ref[...],
                            (((0,), (0,)), ((), ())),
                            preferred_element_type=jnp.float32)
        h = jnp.maximum(h + b1_ref[...], 0.0)
        for k in range(K):
            r = jnp.dot(h, w2_ref[k], preferred_element_type=jnp.float32)
            if k == 0:
                r = r + b2_ref[...]
            out_ref[k] = r

    return pl.pallas_call(
        body,
        grid=(grid,),
        in_specs=[
            pl.BlockSpec((cin, bn), lambda i: (0, i)),
            pl.BlockSpec((cin, CH), lambda i: (0, 0)),
            pl.BlockSpec((1, CH), lambda i: (0, 0)),
            pl.BlockSpec((K, CH, CH), lambda i: (0, 0, 0)),
            pl.BlockSpec((1, CH), lambda i: (0, 0)),
        ],
        out_specs=pl.BlockSpec((K, bn, CH), lambda i: (0, i, 0)),
        out_shape=jax.ShapeDtypeStruct((K, npd, CH), jnp.float32),
    )(g1t, w1p, b1.reshape(1, CH), w2, b2.reshape(1, CH))


def kernel(x, adj, perm, W1, b1, W2, b2):
    xT = x[0].T
    spread = (jnp.arange((NP - N) * K, dtype=jnp.int32) % N).reshape(-1, K)
    adjp = jnp.concatenate([adj, spread], axis=0)
    idx1 = adjp.reshape(-1)
    g1 = _sc_gather_x3(xT, idx1, name="sc_gather_x")
    g1t = g1.reshape(K * 3, NP)
    t2 = _tc_convs(g1t, W1.reshape(K * 3, CH), b1, W2, b2)
    t2f = t2.reshape(K * NP, CH)
    idx2 = (adjp + NP * jnp.arange(K, dtype=jnp.int32)[None, :]).reshape(-1)
    out2 = _sc_gather_group(t2f, idx2, grp=K, iw=96, spb=3, nch=50,
                            relu=True, scale=None, mask_n=N,
                            name="sc_conv2_gsum")
    permc = jnp.where(perm < N, perm, N + (perm - N) % (NP - N))
    pooled = _sc_gather_group(out2, permc.astype(jnp.int32), grp=4, iw=64,
                              spb=4, nch=8, relu=False, scale=0.25,
                              mask_n=None, name="sc_pool")
    return pooled[None]
